# baseline (speedup 1.0000x reference)
import functools

import jax
import jax.numpy as jnp
import numpy as np
from jax import lax
from jax.experimental import pallas as pl
from jax.experimental.pallas import tpu as pltpu
from jax.experimental.pallas import tpu_sc as plsc

_ALPHA = 0.1
_IMIN = np.int32(-(2**31))
_NPAD = 80


def _sc_mine(am_hbm, bias_hbm, out_hbm, ambuf, biasv, outv, *, n, nsamp):
    nn = n * n
    ngroups = _NPAD // 16
    wid = lax.axis_index("s") * 2 + lax.axis_index("c")
    pltpu.sync_copy(bias_hbm, biasv)
    iota = lax.iota(jnp.int32, 16)
    kmax0 = jnp.full((16,), _IMIN, jnp.int32)
    kmin0 = jnp.full((16,), np.int32(2**31 - 1), jnp.int32)

    def chunk_body(cc, carry):
        cbase = (wid * (nsamp // (32 * 8)) + cc) * 8
        pltpu.sync_copy(am_hbm.at[pl.ds(cbase * nn, 8 * nn)], ambuf)
        for ls in range(8):
            for g in range(ngroups):
                ibase = g * 16
                rows = iota + ibase
                base = iota * n + (ls * nn + ibase * n)
                gkw = {} if ibase + 16 <= n else {"mask": rows < n}

                def col_body(j, kc, rows=rows, base=base, gkw=gkw):
                    cmax, cmin = kc
                    cidx = jnp.full((16,), j, jnp.int32)
                    v = plsc.load_gather(ambuf, [base + cidx], **gkw)
                    t = jnp.bitwise_and(plsc.bitcast(v, jnp.int32),
                                        np.int32(~127))
                    bm = plsc.load_gather(biasv, [cidx * _NPAD + rows])
                    cmax = jnp.maximum(cmax, t + (127 - cidx) + bm)
                    bn = lax.shift_right_logical(bm, 1)
                    cmin = jnp.minimum(cmin, t + cidx + bn)
                    return cmax, cmin

                cmax, cmin = lax.fori_loop(0, n, col_body, (kmax0, kmin0),
                                           unroll=5)
                jmax16 = 127 - jnp.bitwise_and(cmax, 127)
                jmin16 = jnp.bitwise_and(cmin, 127)
                outv[pl.ds(ls * _NPAD + ibase, 16)] = (
                    jnp.left_shift(jmax16, 8) | jmin16)
        pltpu.sync_copy(outv, out_hbm.at[pl.ds(cbase * _NPAD, 8 * _NPAD)])
        return carry

    lax.fori_loop(0, nsamp // (32 * 8), chunk_body, 0)


def _tc_body(x_ref, code_ref, o_ref, *, bb, n, total_count):
    b = pl.program_id(0)
    nb = pl.num_programs(0)
    code = code_ref[...][:, :n]
    jmax = lax.shift_right_logical(code, 8)
    jmin = jnp.bitwise_and(code, 255)

    xall = x_ref[...]
    xb = xall.astype(jnp.bfloat16)
    r2 = jnp.sum(xall * xall, axis=2, keepdims=True)
    ones = jnp.ones((bb, n, 1), jnp.bfloat16)
    xa = jnp.concatenate([xb, ones], axis=2)
    ya = jnp.concatenate([-2.0 * xb, r2.astype(jnp.bfloat16)], axis=2)
    u = jax.lax.dot_general(xa, ya, (((2,), (2,)), ((0,), (0,))),
                            preferred_element_type=jnp.float32)
    jidx = jax.lax.broadcasted_iota(jnp.int32, (bb, n, n), 2)
    sp = jnp.sum(jnp.where(jidx == jmax[:, :, None], u, 0.0), axis=2)
    sn = jnp.sum(jnp.where(jidx == jmin[:, :, None], u, 0.0), axis=2)
    r2f = r2.reshape(bb, n)
    dp = jnp.sqrt(jnp.maximum(r2f + sp, 0.0))
    dn = jnp.sqrt(jnp.maximum(r2f + sn, 0.0))
    h = jnp.maximum(dp - dn + _ALPHA, 0.0)

    acc = jnp.where(b == 0, h, o_ref[...] + h)
    mean_bcast = jnp.full((bb, n), jnp.sum(acc) / total_count, jnp.float32)
    o_ref[...] = jnp.where(b == nb - 1, mean_bcast, acc)


@jax.jit
def kernel(x, annotator_matrix, num_dist_types, num_levels):
    b, n, d = x.shape
    m = n - 1
    i = jnp.arange(n)[:, None]
    j = jnp.arange(_NPAD)[None, :]
    same_block = ((j - 1) // num_levels) == (((i - 1) * num_dist_types) // m)
    valid = jnp.where(j == 0, i > 0, jnp.where(i == 0, True, ~same_block))
    valid = valid & (j < n)
    bias = jnp.where(valid, np.int32(0), _IMIN).astype(jnp.int32)

    mesh = plsc.VectorSubcoreMesh(core_axis_name="c", subcore_axis_name="s")
    mine = pl.kernel(
        functools.partial(_sc_mine, n=n, nsamp=b),
        out_type=jax.ShapeDtypeStruct((b * _NPAD,), jnp.int32),
        mesh=mesh,
        compiler_params=pltpu.CompilerParams(needs_layout_passes=False),
        scratch_types=[
            pltpu.VMEM((8 * n * n,), jnp.float32),
            pltpu.VMEM((n * _NPAD,), jnp.int32),
            pltpu.VMEM((8 * _NPAD,), jnp.int32),
        ],
    )
    code = mine(annotator_matrix.reshape(-1), bias.reshape(-1)).reshape(b, _NPAD)

    bb = 16
    grid = b // bb
    out = pl.pallas_call(
        functools.partial(_tc_body, bb=bb, n=n, total_count=b * n),
        grid=(grid,),
        in_specs=[
            pl.BlockSpec((bb, n, d), lambda g: (g, 0, 0)),
            pl.BlockSpec((bb, _NPAD), lambda g: (g, 0)),
        ],
        out_specs=pl.BlockSpec((bb, n), lambda g: (0, 0)),
        out_shape=jax.ShapeDtypeStruct((bb, n), jnp.float32),
    )(x, code)
    return out[0, 0]
